# Initial kernel scaffold; baseline (speedup 1.0000x reference)
import jax
import jax.numpy as jnp
from jax.experimental import pallas as pl


def kernel(x, params):
    raise NotImplementedError("write your pallas kernel here")


# speedup vs baseline: 1.0769x; 1.0769x over previous
import functools
import math

import jax
import jax.numpy as jnp
from jax.experimental import pallas as pl

NEG_INF = float("-inf")
F32 = jnp.float32


def _dot(a, b):
    return jnp.dot(a, b, preferred_element_type=F32)


def _pre_kernel(x_ref, iW1, ib1, iW2, ib2, fW1, fb1, fW2, fb2, mW, mb,
                int_ref, feat_ref, msg_ref):
    x = x_ref[...]
    h = jax.nn.gelu(_dot(x, iW1[...]) + ib1[...])
    int_ref[...] = _dot(h, iW2[...]) + ib2[...]
    h = jax.nn.gelu(_dot(x, fW1[...]) + fb1[...])
    feat_ref[...] = _dot(h, fW2[...]) + fb2[...]
    msg_ref[...] = _dot(x, mW[...]) + mb[...]


def _pre_call(cur, p, mp, bm):
    S, D = cur.shape
    H = D // 2
    full = lambda shp: pl.BlockSpec(shp, lambda i: (0,) * len(shp))
    row = pl.BlockSpec((bm, D), lambda i: (i, 0))
    return pl.pallas_call(
        _pre_kernel,
        grid=(S // bm,),
        in_specs=[row,
                  full((D, H)), full((1, H)), full((H, D)), full((1, D)),
                  full((D, H)), full((1, H)), full((H, D)), full((1, D)),
                  full((D, D)), full((1, D))],
        out_specs=(row, row, row),
        out_shape=(jax.ShapeDtypeStruct((S, D), F32),) * 3,
    )(cur, p["iW1"], p["ib1"].reshape(1, H), p["iW2"], p["ib2"].reshape(1, D),
      p["fW1"], p["fb1"].reshape(1, H), p["fW2"], p["fb2"].reshape(1, D),
      mp["mW"], mp["mb"].reshape(1, D))


def _attn_kernel(kk, scale, int_ref, feat_ref, msg_ref, lmask_ref, out_ref):
    s = jax.lax.dot_general(int_ref[...], feat_ref[...],
                            (((1,), (1,)), ((), ())),
                            preferred_element_type=F32)
    s = s * scale + lmask_ref[...]
    rowmax = jnp.max(s, axis=-1, keepdims=True)

    def body(_, t):
        m = jnp.max(t, axis=-1, keepdims=True)
        return jnp.where(t >= m, NEG_INF, t)

    t = jax.lax.fori_loop(0, kk - 1, body, s)
    thr = jnp.max(t, axis=-1, keepdims=True)
    w = jnp.where(s >= thr, jnp.exp(s - rowmax), 0.0)
    p = w / jnp.sum(w, axis=-1, keepdims=True)
    out_ref[...] = _dot(p, msg_ref[...])


def _attn_call(intents, feats, msgs, lmask, kk, bm):
    S, D = intents.shape
    scale = 1.0 / math.sqrt(D)
    full = lambda shp: pl.BlockSpec(shp, lambda i: (0,) * len(shp))
    row = pl.BlockSpec((bm, D), lambda i: (i, 0))
    return pl.pallas_call(
        functools.partial(_attn_kernel, kk, scale),
        grid=(S // bm,),
        in_specs=[row, full((S, D)), full((S, D)),
                  pl.BlockSpec((bm, S), lambda i: (i, 0))],
        out_specs=row,
        out_shape=jax.ShapeDtypeStruct((S, D), F32),
    )(intents, feats, msgs, lmask)


def _gate_kernel(x_ref, agg_ref, aW, ab, gWx, gWa, gb, oW, ob, out_ref):
    x = x_ref[...]
    a2 = _dot(agg_ref[...], aW[...]) + ab[...]
    g = jax.nn.sigmoid(_dot(x, gWx[...]) + _dot(a2, gWa[...]) + gb[...])
    u = x * (1.0 - g) + a2 * g
    out_ref[...] = _dot(u, oW[...]) + ob[...]


def _gate_call(cur, agg, mp, bm):
    S, D = cur.shape
    full = lambda shp: pl.BlockSpec(shp, lambda i: (0,) * len(shp))
    row = pl.BlockSpec((bm, D), lambda i: (i, 0))
    return pl.pallas_call(
        _gate_kernel,
        grid=(S // bm,),
        in_specs=[row, row,
                  full((D, D)), full((1, D)),
                  full((D, D)), full((D, D)), full((1, D)),
                  full((D, D)), full((1, D))],
        out_specs=row,
        out_shape=jax.ShapeDtypeStruct((S, D), F32),
    )(cur, agg, mp["aW"], mp["ab"].reshape(1, D),
      mp["gW"][:D], mp["gW"][D:], mp["gb"].reshape(1, D),
      mp["oW"], mp["ob"].reshape(1, D))


def _blockagg_kernel(xr_ref, W, b, out_ref):
    m = jnp.mean(xr_ref[...], axis=1)
    out_ref[...] = jax.nn.gelu(_dot(m, W[...]) + b[...])


def _blockagg_call(xr, W, b, bm):
    nb, bs, D = xr.shape
    full = lambda shp: pl.BlockSpec(shp, lambda i: (0,) * len(shp))
    return pl.pallas_call(
        _blockagg_kernel,
        grid=(nb // bm,),
        in_specs=[pl.BlockSpec((bm, bs, D), lambda i: (i, 0, 0)),
                  full((D, D)), full((1, D))],
        out_specs=pl.BlockSpec((bm, D), lambda i: (i, 0)),
        out_shape=jax.ShapeDtypeStruct((nb, D), F32),
    )(xr, W, b.reshape(1, D))


def _linear_kernel(x_ref, W, b, out_ref):
    out_ref[...] = _dot(x_ref[...], W[...]) + b[...]


def _linear_call(x, W, b, bm):
    S, D = x.shape
    full = lambda shp: pl.BlockSpec(shp, lambda i: (0,) * len(shp))
    row = pl.BlockSpec((bm, D), lambda i: (i, 0))
    return pl.pallas_call(
        _linear_kernel,
        grid=(S // bm,),
        in_specs=[row, full((D, D)), full((1, D))],
        out_specs=row,
        out_shape=jax.ShapeDtypeStruct((S, D), F32),
    )(x, W, b.reshape(1, D))


def _final_kernel(o0_ref, d1_ref, d2_ref, F0, F1, F2, fb, out_ref):
    acc = _dot(o0_ref[...], F0[...])
    acc = acc + _dot(d1_ref[...], F1[...])
    acc = acc + _dot(d2_ref[...], F2[...])
    out_ref[...] = acc + fb[...]


def _final_call(o0, d1, d2, fW, fb, bm):
    S, D = o0.shape
    full = lambda shp: pl.BlockSpec(shp, lambda i: (0,) * len(shp))
    row = pl.BlockSpec((bm, D), lambda i: (i, 0))
    return pl.pallas_call(
        _final_kernel,
        grid=(S // bm,),
        in_specs=[row, row, row,
                  full((D, D)), full((D, D)), full((D, D)), full((1, D))],
        out_specs=row,
        out_shape=jax.ShapeDtypeStruct((S, D), F32),
    )(o0, d1, d2, fW[:D], fW[D:2 * D], fW[2 * D:], fb.reshape(1, D))


def _locality_mask(S, lb):
    pos = jnp.arange(S)
    rel = pos[None, :] - pos[:, None]
    L = lb.shape[-1]
    rc = jnp.clip(rel, -(L // 2), L // 2 - 1) + L // 2
    return jnp.where(rel > 0, NEG_INF, lb[rc]).astype(F32)


BLOCK_SIZES_ = (1, 4, 16)


def kernel(x, params):
    xb = x[0]
    S, D = xb.shape
    outs = []
    for level, bs in enumerate(BLOCK_SIZES_):
        if bs == 1:
            cur = xb
        else:
            ap = params["block_agg"][level - 1]
            cur = _blockagg_call(xb.reshape(S // bs, bs, D), ap["W"], ap["b"],
                                 bm=min(256, S // bs))
        Sl = cur.shape[0]
        bm = min(256, Sl)
        p = params["pred"][level]
        intents, feats, msgs = _pre_call(cur, p, params["mp"][level], bm)
        lmask = _locality_mask(Sl, p["lb"])
        agg = _attn_call(intents, feats, msgs, lmask, 32 // (2 ** level), bm)
        lo = _gate_call(cur, agg, params["mp"][level], bm)
        if bs > 1:
            dp = params["block_dist"][level - 1]
            dist = _linear_call(lo, dp["W"], dp["b"], bm)
            outs.append(jnp.repeat(dist, bs, axis=0))
        else:
            outs.append(lo)
    out = _final_call(outs[0], outs[1], outs[2], params["fW"], params["fb"],
                      bm=256)
    return out[None]


# speedup vs baseline: 93.5581x; 86.8776x over previous
import functools
import math

import jax
import jax.numpy as jnp
from jax.experimental import pallas as pl

NEG_INF = float("-inf")
F32 = jnp.float32


def _dot(a, b):
    return jnp.dot(a, b, preferred_element_type=F32)


def _pre_kernel(x_ref, iW1, ib1, iW2, ib2, fW1, fb1, fW2, fb2, mW, mb,
                int_ref, feat_ref, msg_ref):
    x = x_ref[...]
    h = jax.nn.gelu(_dot(x, iW1[...]) + ib1[...])
    int_ref[...] = _dot(h, iW2[...]) + ib2[...]
    h = jax.nn.gelu(_dot(x, fW1[...]) + fb1[...])
    feat_ref[...] = _dot(h, fW2[...]) + fb2[...]
    msg_ref[...] = _dot(x, mW[...]) + mb[...]


def _pre_call(cur, p, mp, bm):
    S, D = cur.shape
    H = D // 2
    full = lambda shp: pl.BlockSpec(shp, lambda i: (0,) * len(shp))
    row = pl.BlockSpec((bm, D), lambda i: (i, 0))
    return pl.pallas_call(
        _pre_kernel,
        grid=(S // bm,),
        in_specs=[row,
                  full((D, H)), full((1, H)), full((H, D)), full((1, D)),
                  full((D, H)), full((1, H)), full((H, D)), full((1, D)),
                  full((D, D)), full((1, D))],
        out_specs=(row, row, row),
        out_shape=(jax.ShapeDtypeStruct((S, D), F32),) * 3,
    )(cur, p["iW1"], p["ib1"].reshape(1, H), p["iW2"], p["ib2"].reshape(1, D),
      p["fW1"], p["fb1"].reshape(1, H), p["fW2"], p["fb2"].reshape(1, D),
      mp["mW"], mp["mb"].reshape(1, D))


def _attn_kernel(kk, scale, int_ref, feat_ref, msg_ref, lmask_ref, out_ref):
    s = jax.lax.dot_general(int_ref[...], feat_ref[...],
                            (((1,), (1,)), ((), ())),
                            preferred_element_type=F32)
    s = s * scale + lmask_ref[...]
    rowmax = jnp.max(s, axis=-1, keepdims=True)

    def body(_, t):
        m = jnp.max(t, axis=-1, keepdims=True)
        return jnp.where(t >= m, NEG_INF, t)

    t = jax.lax.fori_loop(0, kk - 1, body, s)
    thr = jnp.max(t, axis=-1, keepdims=True)
    w = jnp.where(s >= thr, jnp.exp(s - rowmax), 0.0)
    p = w / jnp.sum(w, axis=-1, keepdims=True)
    out_ref[...] = _dot(p, msg_ref[...])


def _attn_call(intents, feats, msgs, lmask, kk, bm):
    S, D = intents.shape
    scale = 1.0 / math.sqrt(D)
    full = lambda shp: pl.BlockSpec(shp, lambda i: (0,) * len(shp))
    row = pl.BlockSpec((bm, D), lambda i: (i, 0))
    return pl.pallas_call(
        functools.partial(_attn_kernel, kk, scale),
        grid=(S // bm,),
        in_specs=[row, full((S, D)), full((S, D)),
                  pl.BlockSpec((bm, S), lambda i: (i, 0))],
        out_specs=row,
        out_shape=jax.ShapeDtypeStruct((S, D), F32),
    )(intents, feats, msgs, lmask)


def _gate_kernel(x_ref, agg_ref, aW, ab, gWx, gWa, gb, oW, ob, out_ref):
    x = x_ref[...]
    a2 = _dot(agg_ref[...], aW[...]) + ab[...]
    g = jax.nn.sigmoid(_dot(x, gWx[...]) + _dot(a2, gWa[...]) + gb[...])
    u = x * (1.0 - g) + a2 * g
    out_ref[...] = _dot(u, oW[...]) + ob[...]


def _gate_call(cur, agg, mp, bm):
    S, D = cur.shape
    full = lambda shp: pl.BlockSpec(shp, lambda i: (0,) * len(shp))
    row = pl.BlockSpec((bm, D), lambda i: (i, 0))
    return pl.pallas_call(
        _gate_kernel,
        grid=(S // bm,),
        in_specs=[row, row,
                  full((D, D)), full((1, D)),
                  full((D, D)), full((D, D)), full((1, D)),
                  full((D, D)), full((1, D))],
        out_specs=row,
        out_shape=jax.ShapeDtypeStruct((S, D), F32),
    )(cur, agg, mp["aW"], mp["ab"].reshape(1, D),
      mp["gW"][:D], mp["gW"][D:], mp["gb"].reshape(1, D),
      mp["oW"], mp["ob"].reshape(1, D))


def _blockagg_kernel(xr_ref, W, b, out_ref):
    m = jnp.mean(xr_ref[...], axis=1)
    out_ref[...] = jax.nn.gelu(_dot(m, W[...]) + b[...])


def _blockagg_call(xr, W, b, bm):
    nb, bs, D = xr.shape
    full = lambda shp: pl.BlockSpec(shp, lambda i: (0,) * len(shp))
    return pl.pallas_call(
        _blockagg_kernel,
        grid=(nb // bm,),
        in_specs=[pl.BlockSpec((bm, bs, D), lambda i: (i, 0, 0)),
                  full((D, D)), full((1, D))],
        out_specs=pl.BlockSpec((bm, D), lambda i: (i, 0)),
        out_shape=jax.ShapeDtypeStruct((nb, D), F32),
    )(xr, W, b.reshape(1, D))


def _linear_kernel(x_ref, W, b, out_ref):
    out_ref[...] = _dot(x_ref[...], W[...]) + b[...]


def _linear_call(x, W, b, bm):
    S, D = x.shape
    full = lambda shp: pl.BlockSpec(shp, lambda i: (0,) * len(shp))
    row = pl.BlockSpec((bm, D), lambda i: (i, 0))
    return pl.pallas_call(
        _linear_kernel,
        grid=(S // bm,),
        in_specs=[row, full((D, D)), full((1, D))],
        out_specs=row,
        out_shape=jax.ShapeDtypeStruct((S, D), F32),
    )(x, W, b.reshape(1, D))


def _final_kernel(o0_ref, d1_ref, d2_ref, F0, F1, F2, fb, out_ref):
    acc = _dot(o0_ref[...], F0[...])
    acc = acc + _dot(d1_ref[...], F1[...])
    acc = acc + _dot(d2_ref[...], F2[...])
    out_ref[...] = acc + fb[...]


def _final_call(o0, d1, d2, fW, fb, bm):
    S, D = o0.shape
    full = lambda shp: pl.BlockSpec(shp, lambda i: (0,) * len(shp))
    row = pl.BlockSpec((bm, D), lambda i: (i, 0))
    return pl.pallas_call(
        _final_kernel,
        grid=(S // bm,),
        in_specs=[row, row, row,
                  full((D, D)), full((D, D)), full((D, D)), full((1, D))],
        out_specs=row,
        out_shape=jax.ShapeDtypeStruct((S, D), F32),
    )(o0, d1, d2, fW[:D], fW[D:2 * D], fW[2 * D:], fb.reshape(1, D))


def _locality_mask(S, lb):
    L = lb.shape[-1]
    t = jnp.arange(2 * S)
    relv = jnp.where(t >= S, t - 2 * S, t)
    idx = jnp.clip(relv, -(L // 2), L // 2 - 1) + L // 2
    v = jax.nn.one_hot(idx, L, dtype=F32) @ lb
    v = jnp.where(relv > 0, NEG_INF, v)
    X = jnp.broadcast_to(v, (S, 2 * S)).reshape(-1)[: S * (2 * S - 1)]
    return X.reshape(S, 2 * S - 1)[:, :S]


BLOCK_SIZES_ = (1, 4, 16)


def kernel(x, params):
    xb = x[0]
    S, D = xb.shape
    outs = []
    for level, bs in enumerate(BLOCK_SIZES_):
        if bs == 1:
            cur = xb
        else:
            ap = params["block_agg"][level - 1]
            cur = _blockagg_call(xb.reshape(S // bs, bs, D), ap["W"], ap["b"],
                                 bm=min(256, S // bs))
        Sl = cur.shape[0]
        bm = min(256, Sl)
        p = params["pred"][level]
        intents, feats, msgs = _pre_call(cur, p, params["mp"][level], bm)
        lmask = _locality_mask(Sl, p["lb"])
        agg = _attn_call(intents, feats, msgs, lmask, 32 // (2 ** level), bm)
        lo = _gate_call(cur, agg, params["mp"][level], bm)
        if bs > 1:
            dp = params["block_dist"][level - 1]
            dist = _linear_call(lo, dp["W"], dp["b"], bm)
            outs.append(jnp.repeat(dist, bs, axis=0))
        else:
            outs.append(lo)
    out = _final_call(outs[0], outs[1], outs[2], params["fW"], params["fb"],
                      bm=256)
    return out[None]
